# baseline (speedup 1.0000x reference)
import jax
import jax.numpy as jnp
from jax import lax
from jax.experimental import pallas as pl
from jax.experimental.pallas import tpu as pltpu
from jax.experimental.pallas import tpu_sc as plsc

N = 10000
E = 320000
D = 128
DH = D // 2
NP = 10240
EP = 327680
NC = 2
NS = 16
NW = NC * NS
ECH = EP // NW
ETILE = EP // NS
CHUNK = 64
NCHUNK = ETILE // CHUNK
NROW = N // NS
NACC = 10112
RPT = NACC // NS


def _proj_body(x_ref, w_ref, al_ref, ar_ref, f_ref, el_ref, er_ref, c_ref,
               mx_ref):
    i = pl.program_id(0)
    feat = lax.dot_general(x_ref[...], w_ref[...], (((1,), (1,)), ((), ())),
                           preferred_element_type=jnp.float32,
                           precision=lax.Precision.HIGHEST)
    f_ref[0] = feat[:, :DH]
    f_ref[1] = feat[:, DH:]
    el = jnp.sum(feat * al_ref[...], axis=1, keepdims=True)
    er = jnp.sum(feat * ar_ref[...], axis=1, keepdims=True)
    el_ref[...] = el
    er_ref[...] = er

    @pl.when(i == 0)
    def _():
        mx_ref[0] = -jnp.inf
        mx_ref[1] = -jnp.inf

    mx_ref[0] = jnp.maximum(mx_ref[0], jnp.max(el))
    mx_ref[1] = jnp.maximum(mx_ref[1], jnp.max(er))

    @pl.when(i == pl.num_programs(0) - 1)
    def _():
        c = jnp.maximum(mx_ref[0] + mx_ref[1], 0.0)
        c_ref[...] = jnp.full((1, 128), c, jnp.float32)


def _projection(x, W, attn_l, attn_r):
    blk = 1000
    grid = (N // blk,)
    return pl.pallas_call(
        _proj_body,
        grid=grid,
        in_specs=[
            pl.BlockSpec((blk, D), lambda i: (i, 0)),
            pl.BlockSpec((D, D), lambda i: (0, 0)),
            pl.BlockSpec((1, D), lambda i: (0, 0)),
            pl.BlockSpec((1, D), lambda i: (0, 0)),
        ],
        out_specs=[
            pl.BlockSpec((NC, blk, DH), lambda i: (0, i, 0)),
            pl.BlockSpec((blk, 1), lambda i: (i, 0)),
            pl.BlockSpec((blk, 1), lambda i: (i, 0)),
            pl.BlockSpec((1, D), lambda i: (0, 0)),
        ],
        out_shape=[
            jax.ShapeDtypeStruct((NC, N, DH), jnp.float32),
            jax.ShapeDtypeStruct((N, 1), jnp.float32),
            jax.ShapeDtypeStruct((N, 1), jnp.float32),
            jax.ShapeDtypeStruct((1, D), jnp.float32),
        ],
        scratch_shapes=[pltpu.SMEM((2,), jnp.float32)],
    )(x, W, attn_l.reshape(1, D), attn_r.reshape(1, D))


def _edge_body(src_hbm, dst_hbm, el_hbm, er_hbm, c_hbm,
               p_hbm, pk_hbm, dpart_hbm,
               src_v, dst_v, el_v, er_v, p_v, pk_v, den_v, c_v):
    wid = lax.axis_index("s") * NC + lax.axis_index("c")
    base = wid * ECH
    pltpu.sync_copy(src_hbm.at[pl.ds(base, ECH)], src_v)
    pltpu.sync_copy(dst_hbm.at[pl.ds(base, ECH)], dst_v)
    pltpu.sync_copy(el_hbm, el_v)
    pltpu.sync_copy(er_hbm, er_v)
    pltpu.sync_copy(c_hbm, c_v)
    cv = c_v[...]

    def zero_body(i, _):
        den_v[pl.ds(i * 16, 16)] = jnp.zeros((16,), jnp.float32)
        return 0

    lax.fori_loop(0, NP // 16, zero_body, 0)

    def edge_body(j, _):
        sl = pl.ds(j * 16, 16)
        sv = src_v[sl]
        dv = dst_v[sl]
        a = plsc.load_gather(el_v, [sv])
        b = plsc.load_gather(er_v, [dv])
        u = a + b
        e = jnp.where(u >= 0.0, u, u * jnp.float32(0.2))
        pe = jnp.exp(e - cv)
        p_v[sl] = pe
        pk_v[sl] = jnp.bitwise_or(lax.shift_left(dv, 16), sv)
        plsc.addupdate_scatter(den_v, [dv], pe)
        return 0

    lax.fori_loop(0, ECH // 16, edge_body, 0)
    pltpu.sync_copy(p_v, p_hbm.at[pl.ds(base, ECH)])
    pltpu.sync_copy(pk_v, pk_hbm.at[pl.ds(base, ECH)])
    pltpu.sync_copy(den_v, dpart_hbm.at[wid])


_edge_kernel = pl.kernel(
    _edge_body,
    out_type=[
        jax.ShapeDtypeStruct((EP,), jnp.float32),
        jax.ShapeDtypeStruct((EP,), jnp.int32),
        jax.ShapeDtypeStruct((NW, NP), jnp.float32),
    ],
    mesh=plsc.VectorSubcoreMesh(core_axis_name="c", subcore_axis_name="s"),
    scratch_types=[
        pltpu.VMEM((ECH,), jnp.int32),
        pltpu.VMEM((ECH,), jnp.int32),
        pltpu.VMEM((NP,), jnp.float32),
        pltpu.VMEM((NP,), jnp.float32),
        pltpu.VMEM((ECH,), jnp.float32),
        pltpu.VMEM((ECH,), jnp.int32),
        pltpu.VMEM((NP,), jnp.float32),
        pltpu.VMEM((16,), jnp.float32),
    ],
    compiler_params=pltpu.CompilerParams(needs_layout_passes=False),
)


def _den_body(d_ref, dinv_ref):
    s = jnp.sum(d_ref[...], axis=0, keepdims=True)
    dinv_ref[...] = 1.0 / jnp.maximum(s, 1e-9)


def _den_reduce(dpart):
    return pl.pallas_call(
        _den_body,
        out_shape=jax.ShapeDtypeStruct((1, NP), jnp.float32),
    )(dpart)


def _agg_body(pk_hbm, p_hbm, fh_hbm,
              out_hbm,
              pk_v, p_v, sia, sib, dia, dib, ra, rb,
              semf_a, semf_b, sems_a, sems_b, feat_sp, acc):
    cid = lax.axis_index("c")
    sid = lax.axis_index("s")
    ebase = sid * ETILE
    pltpu.sync_copy(pk_hbm.at[pl.ds(ebase, ETILE)], pk_v)
    pltpu.sync_copy(p_hbm.at[pl.ds(ebase, ETILE)], p_v)

    pltpu.sync_copy(fh_hbm.at[cid].at[pl.ds(sid * NROW, NROW)],
                    feat_sp.at[pl.ds(sid * NROW, NROW)])

    def zrow(i, _):
        def zcol(c, _):
            ra[i, pl.ds(c * 16, 16)] = jnp.zeros((16,), jnp.float32)
            return 0
        lax.fori_loop(0, DH // 16, zcol, 0)
        return 0

    lax.fori_loop(0, CHUNK, zrow, 0)
    for z in range(RPT // CHUNK):
        pltpu.sync_copy(ra, acc.at[pl.ds(sid * RPT + z * CHUNK, CHUNK)])
    rem = RPT % CHUNK
    if rem:
        pltpu.sync_copy(ra.at[pl.ds(0, rem)],
                        acc.at[pl.ds(sid * RPT + (RPT // CHUNK) * CHUNK, rem)])
    plsc.subcore_barrier()

    def unpack(j, sibuf, dibuf):
        def ugrp(c, _):
            sl = pl.ds(c * 16, 16)
            pk = pk_v[pl.ds(j * CHUNK + c * 16, 16)]
            sibuf[sl] = jnp.bitwise_and(pk, 0xFFFF)
            dibuf[sl] = lax.shift_right_logical(pk, 16)
            return 0

        lax.fori_loop(0, CHUNK // 16, ugrp, 0)

    def gather_hbm(sibuf, rbuf, semf):
        pltpu.async_copy(fh_hbm.at[cid].at[sibuf], rbuf, semf)

    def gather_a(sibuf, rbuf, semf):
        gather_hbm(sibuf, rbuf, semf)

    def gwait_a(sibuf, rbuf, semf):
        pltpu.make_async_copy(fh_hbm.at[cid].at[sibuf], rbuf, semf).wait()

    def gather_b(sibuf, rbuf, semf):
        @pl.when(cid == 0)
        def _():
            gather_hbm(sibuf, rbuf, semf)

        @pl.when(cid != 0)
        def _():
            pltpu.async_copy(feat_sp.at[sibuf], rbuf, semf)

    def gwait_b(sibuf, rbuf, semf):
        @pl.when(cid == 0)
        def _():
            pltpu.make_async_copy(fh_hbm.at[cid].at[sibuf], rbuf, semf).wait()

        @pl.when(cid != 0)
        def _():
            pltpu.make_async_copy(feat_sp.at[sibuf], rbuf, semf).wait()

    def process(j, rbuf, dibuf, sems):
        def cgrp(c, _):
            al = p_v[pl.ds(j * CHUNK + c * 16, 16)]
            for u in range(16):
                r = c * 16 + u
                a = al[u]
                for g in range(DH // 16):
                    sg = pl.ds(g * 16, 16)
                    rbuf[r, sg] = rbuf[r, sg] * a
            return 0

        lax.fori_loop(0, CHUNK // 16, cgrp, 0)
        pltpu.async_copy(rbuf, acc.at[dibuf], sems, add=True)

    def swait(rbuf, dibuf, sems):
        pltpu.make_async_copy(rbuf, acc.at[dibuf], sems).wait()

    unpack(0, sia, dia)
    gather_a(sia, ra, semf_a)

    def pair(jj, _):
        ja = 2 * jj
        jb = ja + 1

        @pl.when(jj > 0)
        def _():
            swait(rb, dib, sems_b)

        unpack(jb, sib, dib)
        gather_b(sib, rb, semf_b)
        gwait_a(sia, ra, semf_a)
        process(ja, ra, dia, sems_a)

        @pl.when(jj < NCHUNK // 2 - 1)
        def _():
            swait(ra, dia, sems_a)
            unpack(ja + 2, sia, dia)
            gather_a(sia, ra, semf_a)

        gwait_b(sib, rb, semf_b)
        process(jb, rb, dib, sems_b)
        return 0

    lax.fori_loop(0, NCHUNK // 2, pair, 0)
    swait(ra, dia, sems_a)
    swait(rb, dib, sems_b)
    plsc.subcore_barrier()

    pltpu.sync_copy(acc.at[pl.ds(sid * RPT, RPT)],
                    out_hbm.at[cid].at[pl.ds(sid * RPT, RPT)])


_agg_kernel = pl.kernel(
    _agg_body,
    out_type=[jax.ShapeDtypeStruct((NC, NACC, DH), jnp.float32)],
    mesh=plsc.VectorSubcoreMesh(core_axis_name="c", subcore_axis_name="s"),
    scratch_types=[
        pltpu.VMEM((ETILE,), jnp.int32),
        pltpu.VMEM((ETILE,), jnp.float32),
        pltpu.VMEM((CHUNK,), jnp.int32),
        pltpu.VMEM((CHUNK,), jnp.int32),
        pltpu.VMEM((CHUNK,), jnp.int32),
        pltpu.VMEM((CHUNK,), jnp.int32),
        pltpu.VMEM((CHUNK, DH), jnp.float32),
        pltpu.VMEM((CHUNK, DH), jnp.float32),
        pltpu.SemaphoreType.DMA,
        pltpu.SemaphoreType.DMA,
        pltpu.SemaphoreType.DMA,
        pltpu.SemaphoreType.DMA,
        pltpu.VMEM_SHARED((N, DH), jnp.float32),
        pltpu.VMEM_SHARED((NACC, DH), jnp.float32),
    ],
    compiler_params=pltpu.CompilerParams(needs_layout_passes=False,
                                         use_tc_tiling_on_sc=False),
)


def _epi_body(part_ref, dinv_ref, bias_ref, out_ref):
    rst = jnp.concatenate([part_ref[0], part_ref[1]], axis=1)
    out_ref[...] = jnp.maximum(rst * dinv_ref[...] + bias_ref[...], 0.0)


def _epilogue(parts, dinv_col, bias):
    blk = 1000
    return pl.pallas_call(
        _epi_body,
        grid=(N // blk,),
        in_specs=[
            pl.BlockSpec((NC, blk, DH), lambda i: (0, i, 0)),
            pl.BlockSpec((blk, 1), lambda i: (i, 0)),
            pl.BlockSpec((1, D), lambda i: (0, 0)),
        ],
        out_specs=pl.BlockSpec((blk, D), lambda i: (i, 0)),
        out_shape=jax.ShapeDtypeStruct((N, D), jnp.float32),
    )(parts, dinv_col, bias.reshape(1, D))


def kernel(x, edge_index, W, attn_l, attn_r, bias):
    src = edge_index[0].astype(jnp.int32)
    dst = edge_index[1].astype(jnp.int32)
    src_p = jnp.pad(src, (0, EP - E))
    dst_p = jnp.pad(dst, (0, EP - E), constant_values=N)

    fhalves, el, er, cmat = _projection(x, W, attn_l, attn_r)
    el_p = jnp.pad(el[:, 0], (0, NP - N))
    er_p = jnp.pad(er[:, 0], (0, NP - N))
    c16 = cmat[0, :16]

    p, pk, dpart = _edge_kernel(src_p, dst_p, el_p, er_p, c16)
    dinv_col = _den_reduce(dpart).reshape(NP, 1)

    (parts,) = _agg_kernel(pk, p, fhalves)
    return _epilogue(parts, dinv_col, bias)


# speedup vs baseline: 1.3432x; 1.3432x over previous
import jax
import jax.numpy as jnp
import numpy as np
from jax import lax
from jax.experimental import pallas as pl
from jax.experimental.pallas import tpu as pltpu
from jax.experimental.pallas import tpu_sc as plsc

N = 10000
E = 320000
D = 128
DH = D // 2
NP = 10240
EP = 327680
NC = 2
NS = 16
NW = NC * NS
ECH = EP // NW
ETILE = EP // NS
CHUNK = 64
NCHUNK = ETILE // CHUNK
NROW = N // NS
NACC = 10112
RPT = NACC // NS


def _proj_body(x_ref, w_ref, al_ref, ar_ref, f_ref, el_ref, er_ref, c_ref,
               mx_ref):
    i = pl.program_id(0)
    feat = lax.dot_general(x_ref[...], w_ref[...], (((1,), (1,)), ((), ())),
                           preferred_element_type=jnp.float32,
                           precision=lax.Precision.HIGHEST)
    f_ref[0] = feat[:, :DH]
    f_ref[1] = feat[:, DH:]
    el = jnp.sum(feat * al_ref[...], axis=1, keepdims=True)
    er = jnp.sum(feat * ar_ref[...], axis=1, keepdims=True)
    el_ref[...] = el
    er_ref[...] = er

    @pl.when(i == 0)
    def _():
        mx_ref[0] = -jnp.inf
        mx_ref[1] = -jnp.inf

    mx_ref[0] = jnp.maximum(mx_ref[0], jnp.max(el))
    mx_ref[1] = jnp.maximum(mx_ref[1], jnp.max(er))

    @pl.when(i == pl.num_programs(0) - 1)
    def _():
        c = jnp.maximum(mx_ref[0] + mx_ref[1], 0.0)
        c_ref[...] = jnp.full((1, 128), c, jnp.float32)


def _projection(x, W, attn_l, attn_r):
    blk = 1000
    grid = (N // blk,)
    return pl.pallas_call(
        _proj_body,
        grid=grid,
        in_specs=[
            pl.BlockSpec((blk, D), lambda i: (i, 0)),
            pl.BlockSpec((D, D), lambda i: (0, 0)),
            pl.BlockSpec((1, D), lambda i: (0, 0)),
            pl.BlockSpec((1, D), lambda i: (0, 0)),
        ],
        out_specs=[
            pl.BlockSpec((NC, blk, DH), lambda i: (0, i, 0)),
            pl.BlockSpec((blk, 1), lambda i: (i, 0)),
            pl.BlockSpec((blk, 1), lambda i: (i, 0)),
            pl.BlockSpec((1, D), lambda i: (0, 0)),
        ],
        out_shape=[
            jax.ShapeDtypeStruct((NC, N, DH), jnp.float32),
            jax.ShapeDtypeStruct((N, 1), jnp.float32),
            jax.ShapeDtypeStruct((N, 1), jnp.float32),
            jax.ShapeDtypeStruct((1, D), jnp.float32),
        ],
        scratch_shapes=[pltpu.SMEM((2,), jnp.float32)],
    )(x, W, attn_l.reshape(1, D), attn_r.reshape(1, D))


def _edge_body(src_hbm, dst_hbm, el_hbm, er_hbm, c_hbm,
               p_hbm, pk_hbm, dpart_hbm,
               src_v, dst_v, el_v, er_v, p_v, pk_v, den_v, c_v):
    wid = lax.axis_index("s") * NC + lax.axis_index("c")
    base = wid * ECH
    pltpu.sync_copy(src_hbm.at[pl.ds(base, ECH)], src_v)
    pltpu.sync_copy(dst_hbm.at[pl.ds(base, ECH)], dst_v)
    pltpu.sync_copy(el_hbm, el_v)
    pltpu.sync_copy(er_hbm, er_v)
    pltpu.sync_copy(c_hbm, c_v)
    cv = c_v[...]

    def zero_body(i, _):
        den_v[pl.ds(i * 16, 16)] = jnp.zeros((16,), jnp.float32)
        return 0

    lax.fori_loop(0, NP // 16, zero_body, 0)

    def edge_body(j, _):
        sl = pl.ds(j * 16, 16)
        sv = src_v[sl]
        dv = dst_v[sl]
        a = plsc.load_gather(el_v, [sv])
        b = plsc.load_gather(er_v, [dv])
        u = a + b
        e = jnp.where(u >= 0.0, u, u * jnp.float32(0.2))
        pe = jnp.exp(e - cv)
        p_v[sl] = pe
        pk_v[sl] = jnp.bitwise_or(lax.shift_left(dv, 16), sv)
        plsc.addupdate_scatter(den_v, [dv], pe)
        return 0

    lax.fori_loop(0, ECH // 16, edge_body, 0)
    pltpu.sync_copy(p_v, p_hbm.at[pl.ds(base, ECH)])
    pltpu.sync_copy(pk_v, pk_hbm.at[pl.ds(base, ECH)])
    pltpu.sync_copy(den_v, dpart_hbm.at[wid])


_edge_kernel = pl.kernel(
    _edge_body,
    out_type=[
        jax.ShapeDtypeStruct((EP,), jnp.float32),
        jax.ShapeDtypeStruct((EP,), jnp.int32),
        jax.ShapeDtypeStruct((NW, NP), jnp.float32),
    ],
    mesh=plsc.VectorSubcoreMesh(core_axis_name="c", subcore_axis_name="s"),
    scratch_types=[
        pltpu.VMEM((ECH,), jnp.int32),
        pltpu.VMEM((ECH,), jnp.int32),
        pltpu.VMEM((NP,), jnp.float32),
        pltpu.VMEM((NP,), jnp.float32),
        pltpu.VMEM((ECH,), jnp.float32),
        pltpu.VMEM((ECH,), jnp.int32),
        pltpu.VMEM((NP,), jnp.float32),
        pltpu.VMEM((16,), jnp.float32),
    ],
    compiler_params=pltpu.CompilerParams(needs_layout_passes=False),
)


def _den_body(d_ref, dinv_ref):
    s = jnp.sum(d_ref[...], axis=0, keepdims=True)
    dinv_ref[...] = 1.0 / jnp.maximum(s, 1e-9)


def _den_reduce(dpart):
    return pl.pallas_call(
        _den_body,
        out_shape=jax.ShapeDtypeStruct((1, NP), jnp.float32),
    )(dpart)


def _agg_body(pk_hbm, p_hbm, fh_hbm,
              out_hbm,
              pk_v, p_v, sia, sib, dia, dib, ra, rb, sa, sb,
              semf_a, semf_b, sems_a, sems_b, feat_sp, acc):
    cid = lax.axis_index("c")
    sid = lax.axis_index("s")
    ebase = sid * ETILE
    pltpu.sync_copy(pk_hbm.at[pl.ds(ebase, ETILE)], pk_v)
    pltpu.sync_copy(p_hbm.at[pl.ds(ebase, ETILE)], p_v)

    pltpu.sync_copy(fh_hbm.at[cid].at[pl.ds(sid * NROW, NROW)],
                    feat_sp.at[pl.ds(sid * NROW, NROW)])

    def zrow(i, _):
        def zcol(c, _):
            sa[i, pl.ds(c * 16, 16)] = jnp.zeros((16,), jnp.float32)
            return 0
        lax.fori_loop(0, DH // 16, zcol, 0)
        return 0

    lax.fori_loop(0, CHUNK, zrow, 0)
    for z in range(RPT // CHUNK):
        pltpu.sync_copy(sa, acc.at[pl.ds(sid * RPT + z * CHUNK, CHUNK)])
    rem = RPT % CHUNK
    if rem:
        pltpu.sync_copy(sa.at[pl.ds(0, rem)],
                        acc.at[pl.ds(sid * RPT + (RPT // CHUNK) * CHUNK, rem)])
    plsc.subcore_barrier()

    def unpack(j, sibuf, dibuf):
        def ugrp(c, _):
            sl = pl.ds(c * 16, 16)
            pk = pk_v[pl.ds(j * CHUNK + c * 16, 16)]
            sibuf[sl] = jnp.bitwise_and(pk, 0xFFFF)
            dibuf[sl] = lax.shift_right_logical(pk, 16)
            return 0

        lax.fori_loop(0, CHUNK // 16, ugrp, 0)

    def gather(sibuf, rbuf, semf):
        pltpu.async_copy(feat_sp.at[sibuf], rbuf, semf)

    def gwait(sibuf, rbuf, semf):
        pltpu.make_async_copy(feat_sp.at[sibuf], rbuf, semf).wait()

    def process(j, rbuf, sbuf, dibuf, sems):
        def cgrp(c, _):
            al = p_v[pl.ds(j * CHUNK + c * 16, 16)]
            for u in range(16):
                r = c * 16 + u
                a = al[u]
                for g in range(DH // 32):
                    row32 = rbuf[r, pl.ds(g * 32, 32)]
                    lo, hi = plsc.unpack(row32,
                                         format=plsc.PackFormat.INTERLEAVED)
                    sbuf[r, pl.ds(g * 32, 16)] = lo * a
                    sbuf[r, pl.ds(g * 32 + 16, 16)] = hi * a
            return 0

        lax.fori_loop(0, CHUNK // 16, cgrp, 0)
        pltpu.async_copy(sbuf, acc.at[dibuf], sems, add=True)

    def swait(sbuf, dibuf, sems):
        pltpu.make_async_copy(sbuf, acc.at[dibuf], sems).wait()

    unpack(0, sia, dia)
    gather(sia, ra, semf_a)

    def pair(jj, _):
        ja = 2 * jj
        jb = ja + 1

        @pl.when(jj > 0)
        def _():
            swait(sb, dib, sems_b)

        unpack(jb, sib, dib)
        gather(sib, rb, semf_b)
        gwait(sia, ra, semf_a)
        process(ja, ra, sa, dia, sems_a)

        @pl.when(jj < NCHUNK // 2 - 1)
        def _():
            swait(sa, dia, sems_a)
            unpack(ja + 2, sia, dia)
            gather(sia, ra, semf_a)

        gwait(sib, rb, semf_b)
        process(jb, rb, sb, dib, sems_b)
        return 0

    lax.fori_loop(0, NCHUNK // 2, pair, 0)
    swait(sa, dia, sems_a)
    swait(sb, dib, sems_b)
    plsc.subcore_barrier()

    pltpu.sync_copy(acc.at[pl.ds(sid * RPT, RPT)],
                    out_hbm.at[cid].at[pl.ds(sid * RPT, RPT)])


_agg_kernel = pl.kernel(
    _agg_body,
    out_type=[jax.ShapeDtypeStruct((NC, NACC, DH), jnp.float32)],
    mesh=plsc.VectorSubcoreMesh(core_axis_name="c", subcore_axis_name="s"),
    scratch_types=[
        pltpu.VMEM((ETILE,), jnp.int32),
        pltpu.VMEM((ETILE,), jnp.float32),
        pltpu.VMEM((CHUNK,), jnp.int32),
        pltpu.VMEM((CHUNK,), jnp.int32),
        pltpu.VMEM((CHUNK,), jnp.int32),
        pltpu.VMEM((CHUNK,), jnp.int32),
        pltpu.VMEM((CHUNK, DH), jnp.bfloat16),
        pltpu.VMEM((CHUNK, DH), jnp.bfloat16),
        pltpu.VMEM((CHUNK, DH), jnp.float32),
        pltpu.VMEM((CHUNK, DH), jnp.float32),
        pltpu.SemaphoreType.DMA,
        pltpu.SemaphoreType.DMA,
        pltpu.SemaphoreType.DMA,
        pltpu.SemaphoreType.DMA,
        pltpu.VMEM_SHARED((N, DH), jnp.bfloat16),
        pltpu.VMEM_SHARED((NACC, DH), jnp.float32),
    ],
    compiler_params=pltpu.CompilerParams(needs_layout_passes=False,
                                         use_tc_tiling_on_sc=False),
)


def _epi_body(part_ref, dinv_ref, bias_ref, out_ref):
    rst = jnp.concatenate([part_ref[0], part_ref[1]], axis=1)
    out_ref[...] = jnp.maximum(rst * dinv_ref[...] + bias_ref[...], 0.0)


def _epilogue(parts, dinv_col, bias):
    blk = 1000
    return pl.pallas_call(
        _epi_body,
        grid=(N // blk,),
        in_specs=[
            pl.BlockSpec((NC, blk, DH), lambda i: (0, i, 0)),
            pl.BlockSpec((blk, 1), lambda i: (i, 0)),
            pl.BlockSpec((1, D), lambda i: (0, 0)),
        ],
        out_specs=pl.BlockSpec((blk, D), lambda i: (i, 0)),
        out_shape=jax.ShapeDtypeStruct((N, D), jnp.float32),
    )(parts, dinv_col, bias.reshape(1, D))


def kernel(x, edge_index, W, attn_l, attn_r, bias):
    src = edge_index[0].astype(jnp.int32)
    dst = edge_index[1].astype(jnp.int32)
    src_p = jnp.pad(src, (0, EP - E))
    dst_p = jnp.pad(dst, (0, EP - E), constant_values=N)

    fhalves, el, er, cmat = _projection(x, W, attn_l, attn_r)
    el_p = jnp.pad(el[:, 0], (0, NP - N))
    er_p = jnp.pad(er[:, 0], (0, NP - N))
    c16 = cmat[0, :16]

    p, pk, dpart = _edge_kernel(src_p, dst_p, el_p, er_p, c16)
    dinv_col = _den_reduce(dpart).reshape(NP, 1)

    perm = np.zeros(DH, dtype=np.int32)
    for g in range(DH // 32):
        for i in range(16):
            perm[32 * g + 2 * i] = 32 * g + i
            perm[32 * g + 2 * i + 1] = 32 * g + 16 + i
    fhbf = fhalves.astype(jnp.bfloat16)[:, :, perm]

    (parts,) = _agg_kernel(pk, p, fhbf)
    return _epilogue(parts, dinv_col, bias)


# speedup vs baseline: 1.3757x; 1.0242x over previous
import jax
import jax.numpy as jnp
import numpy as np
from jax import lax
from jax.experimental import pallas as pl
from jax.experimental.pallas import tpu as pltpu
from jax.experimental.pallas import tpu_sc as plsc

N = 10000
E = 320000
D = 128
DH = D // 2
NP = 10240
EP = 327680
NC = 2
NS = 16
NW = NC * NS
ECH = EP // NW
ETILE = EP // NS
CHUNK = 128
NCHUNK = ETILE // CHUNK
NROW = N // NS
NACC = 10112
RPT = NACC // NS


def _proj_body(x_ref, w_ref, al_ref, ar_ref, f_ref, el_ref, er_ref, c_ref,
               mx_ref):
    i = pl.program_id(0)
    feat = lax.dot_general(x_ref[...], w_ref[...], (((1,), (1,)), ((), ())),
                           preferred_element_type=jnp.float32,
                           precision=lax.Precision.HIGHEST)
    f_ref[0] = feat[:, :DH]
    f_ref[1] = feat[:, DH:]
    el = jnp.sum(feat * al_ref[...], axis=1, keepdims=True)
    er = jnp.sum(feat * ar_ref[...], axis=1, keepdims=True)
    el_ref[...] = el
    er_ref[...] = er

    @pl.when(i == 0)
    def _():
        mx_ref[0] = -jnp.inf
        mx_ref[1] = -jnp.inf

    mx_ref[0] = jnp.maximum(mx_ref[0], jnp.max(el))
    mx_ref[1] = jnp.maximum(mx_ref[1], jnp.max(er))

    @pl.when(i == pl.num_programs(0) - 1)
    def _():
        c = jnp.maximum(mx_ref[0] + mx_ref[1], 0.0)
        c_ref[...] = jnp.full((1, 128), c, jnp.float32)


def _projection(x, W, attn_l, attn_r):
    blk = 1000
    grid = (N // blk,)
    return pl.pallas_call(
        _proj_body,
        grid=grid,
        in_specs=[
            pl.BlockSpec((blk, D), lambda i: (i, 0)),
            pl.BlockSpec((D, D), lambda i: (0, 0)),
            pl.BlockSpec((1, D), lambda i: (0, 0)),
            pl.BlockSpec((1, D), lambda i: (0, 0)),
        ],
        out_specs=[
            pl.BlockSpec((NC, blk, DH), lambda i: (0, i, 0)),
            pl.BlockSpec((blk, 1), lambda i: (i, 0)),
            pl.BlockSpec((blk, 1), lambda i: (i, 0)),
            pl.BlockSpec((1, D), lambda i: (0, 0)),
        ],
        out_shape=[
            jax.ShapeDtypeStruct((NC, N, DH), jnp.float32),
            jax.ShapeDtypeStruct((N, 1), jnp.float32),
            jax.ShapeDtypeStruct((N, 1), jnp.float32),
            jax.ShapeDtypeStruct((1, D), jnp.float32),
        ],
        scratch_shapes=[pltpu.SMEM((2,), jnp.float32)],
    )(x, W, attn_l.reshape(1, D), attn_r.reshape(1, D))


def _edge_body(src_hbm, dst_hbm, el_hbm, er_hbm, c_hbm,
               p_hbm, pk_hbm, dpart_hbm,
               src_v, dst_v, el_v, er_v, p_v, pk_v, den_v, c_v):
    wid = lax.axis_index("s") * NC + lax.axis_index("c")
    base = wid * ECH
    pltpu.sync_copy(src_hbm.at[pl.ds(base, ECH)], src_v)
    pltpu.sync_copy(dst_hbm.at[pl.ds(base, ECH)], dst_v)
    pltpu.sync_copy(el_hbm, el_v)
    pltpu.sync_copy(er_hbm, er_v)
    pltpu.sync_copy(c_hbm, c_v)
    cv = c_v[...]

    def zero_body(i, _):
        den_v[pl.ds(i * 16, 16)] = jnp.zeros((16,), jnp.float32)
        return 0

    lax.fori_loop(0, NP // 16, zero_body, 0)

    def edge_body(j, _):
        sl = pl.ds(j * 16, 16)
        sv = src_v[sl]
        dv = dst_v[sl]
        a = plsc.load_gather(el_v, [sv])
        b = plsc.load_gather(er_v, [dv])
        u = a + b
        e = jnp.where(u >= 0.0, u, u * jnp.float32(0.2))
        pe = jnp.exp(e - cv)
        p_v[sl] = pe
        pk_v[sl] = jnp.bitwise_or(lax.shift_left(dv, 16), sv)
        plsc.addupdate_scatter(den_v, [dv], pe)
        return 0

    lax.fori_loop(0, ECH // 16, edge_body, 0, unroll=2)
    pltpu.sync_copy(p_v, p_hbm.at[pl.ds(base, ECH)])
    pltpu.sync_copy(pk_v, pk_hbm.at[pl.ds(base, ECH)])
    pltpu.sync_copy(den_v, dpart_hbm.at[wid])


_edge_kernel = pl.kernel(
    _edge_body,
    out_type=[
        jax.ShapeDtypeStruct((EP,), jnp.float32),
        jax.ShapeDtypeStruct((EP,), jnp.int32),
        jax.ShapeDtypeStruct((NW, NP), jnp.float32),
    ],
    mesh=plsc.VectorSubcoreMesh(core_axis_name="c", subcore_axis_name="s"),
    scratch_types=[
        pltpu.VMEM((ECH,), jnp.int32),
        pltpu.VMEM((ECH,), jnp.int32),
        pltpu.VMEM((NP,), jnp.float32),
        pltpu.VMEM((NP,), jnp.float32),
        pltpu.VMEM((ECH,), jnp.float32),
        pltpu.VMEM((ECH,), jnp.int32),
        pltpu.VMEM((NP,), jnp.float32),
        pltpu.VMEM((16,), jnp.float32),
    ],
    compiler_params=pltpu.CompilerParams(needs_layout_passes=False),
)


def _den_body(d_ref, dinv_ref):
    s = jnp.sum(d_ref[...], axis=0, keepdims=True)
    dinv_ref[...] = 1.0 / jnp.maximum(s, 1e-9)


def _den_reduce(dpart):
    return pl.pallas_call(
        _den_body,
        out_shape=jax.ShapeDtypeStruct((1, NP), jnp.float32),
    )(dpart)


def _agg_body(pk_hbm, p_hbm, fh_hbm,
              out_hbm,
              pk_v, p_v, sia, sib, dia, dib, ra, rb, sa, sb,
              semf_a, semf_b, sems_a, sems_b, feat_sp, acc):
    cid = lax.axis_index("c")
    sid = lax.axis_index("s")
    ebase = sid * ETILE
    pltpu.sync_copy(pk_hbm.at[pl.ds(ebase, ETILE)], pk_v)
    pltpu.sync_copy(p_hbm.at[pl.ds(ebase, ETILE)], p_v)

    pltpu.sync_copy(fh_hbm.at[cid].at[pl.ds(sid * NROW, NROW)],
                    feat_sp.at[pl.ds(sid * NROW, NROW)])

    def zrow(i, _):
        def zcol(c, _):
            sa[i, pl.ds(c * 16, 16)] = jnp.zeros((16,), jnp.float32)
            return 0
        lax.fori_loop(0, DH // 16, zcol, 0)
        return 0

    lax.fori_loop(0, CHUNK, zrow, 0)
    for z in range(RPT // CHUNK):
        pltpu.sync_copy(sa, acc.at[pl.ds(sid * RPT + z * CHUNK, CHUNK)])
    rem = RPT % CHUNK
    if rem:
        pltpu.sync_copy(sa.at[pl.ds(0, rem)],
                        acc.at[pl.ds(sid * RPT + (RPT // CHUNK) * CHUNK, rem)])
    plsc.subcore_barrier()

    def unpack(j, sibuf, dibuf):
        def ugrp(c, _):
            sl = pl.ds(c * 16, 16)
            pk = pk_v[pl.ds(j * CHUNK + c * 16, 16)]
            sibuf[sl] = jnp.bitwise_and(pk, 0xFFFF)
            dibuf[sl] = lax.shift_right_logical(pk, 16)
            return 0

        lax.fori_loop(0, CHUNK // 16, ugrp, 0)

    def gather(sibuf, rbuf, semf):
        pltpu.async_copy(feat_sp.at[sibuf], rbuf, semf)

    def gwait(sibuf, rbuf, semf):
        pltpu.make_async_copy(feat_sp.at[sibuf], rbuf, semf).wait()

    def process(j, rbuf, sbuf, dibuf, sems):
        def cgrp(c, _):
            al = p_v[pl.ds(j * CHUNK + c * 16, 16)]
            for u in range(16):
                r = c * 16 + u
                a = al[u]
                for g in range(DH // 32):
                    row32 = rbuf[r, pl.ds(g * 32, 32)]
                    lo, hi = plsc.unpack(row32,
                                         format=plsc.PackFormat.INTERLEAVED)
                    sbuf[r, pl.ds(g * 32, 16)] = lo * a
                    sbuf[r, pl.ds(g * 32 + 16, 16)] = hi * a
            return 0

        lax.fori_loop(0, CHUNK // 16, cgrp, 0)
        pltpu.async_copy(sbuf, acc.at[dibuf], sems, add=True)

    def swait(sbuf, dibuf, sems):
        pltpu.make_async_copy(sbuf, acc.at[dibuf], sems).wait()

    unpack(0, sia, dia)
    gather(sia, ra, semf_a)

    def pair(jj, _):
        ja = 2 * jj
        jb = ja + 1

        @pl.when(jj > 0)
        def _():
            swait(sb, dib, sems_b)

        unpack(jb, sib, dib)
        gather(sib, rb, semf_b)
        gwait(sia, ra, semf_a)
        process(ja, ra, sa, dia, sems_a)

        @pl.when(jj < NCHUNK // 2 - 1)
        def _():
            swait(sa, dia, sems_a)
            unpack(ja + 2, sia, dia)
            gather(sia, ra, semf_a)

        gwait(sib, rb, semf_b)
        process(jb, rb, sb, dib, sems_b)
        return 0

    lax.fori_loop(0, NCHUNK // 2, pair, 0)
    swait(sa, dia, sems_a)
    swait(sb, dib, sems_b)
    plsc.subcore_barrier()

    pltpu.sync_copy(acc.at[pl.ds(sid * RPT, RPT)],
                    out_hbm.at[cid].at[pl.ds(sid * RPT, RPT)])


_agg_kernel = pl.kernel(
    _agg_body,
    out_type=[jax.ShapeDtypeStruct((NC, NACC, DH), jnp.float32)],
    mesh=plsc.VectorSubcoreMesh(core_axis_name="c", subcore_axis_name="s"),
    scratch_types=[
        pltpu.VMEM((ETILE,), jnp.int32),
        pltpu.VMEM((ETILE,), jnp.float32),
        pltpu.VMEM((CHUNK,), jnp.int32),
        pltpu.VMEM((CHUNK,), jnp.int32),
        pltpu.VMEM((CHUNK,), jnp.int32),
        pltpu.VMEM((CHUNK,), jnp.int32),
        pltpu.VMEM((CHUNK, DH), jnp.bfloat16),
        pltpu.VMEM((CHUNK, DH), jnp.bfloat16),
        pltpu.VMEM((CHUNK, DH), jnp.float32),
        pltpu.VMEM((CHUNK, DH), jnp.float32),
        pltpu.SemaphoreType.DMA,
        pltpu.SemaphoreType.DMA,
        pltpu.SemaphoreType.DMA,
        pltpu.SemaphoreType.DMA,
        pltpu.VMEM_SHARED((N, DH), jnp.bfloat16),
        pltpu.VMEM_SHARED((NACC, DH), jnp.float32),
    ],
    compiler_params=pltpu.CompilerParams(needs_layout_passes=False,
                                         use_tc_tiling_on_sc=False),
)


def _epi_body(part_ref, dinv_ref, bias_ref, out_ref):
    rst = jnp.concatenate([part_ref[0], part_ref[1]], axis=1)
    out_ref[...] = jnp.maximum(rst * dinv_ref[...] + bias_ref[...], 0.0)


def _epilogue(parts, dinv_col, bias):
    blk = 1000
    return pl.pallas_call(
        _epi_body,
        grid=(N // blk,),
        in_specs=[
            pl.BlockSpec((NC, blk, DH), lambda i: (0, i, 0)),
            pl.BlockSpec((blk, 1), lambda i: (i, 0)),
            pl.BlockSpec((1, D), lambda i: (0, 0)),
        ],
        out_specs=pl.BlockSpec((blk, D), lambda i: (i, 0)),
        out_shape=jax.ShapeDtypeStruct((N, D), jnp.float32),
    )(parts, dinv_col, bias.reshape(1, D))


def kernel(x, edge_index, W, attn_l, attn_r, bias):
    src = edge_index[0].astype(jnp.int32)
    dst = edge_index[1].astype(jnp.int32)
    src_p = jnp.pad(src, (0, EP - E))
    dst_p = jnp.pad(dst, (0, EP - E), constant_values=N)

    fhalves, el, er, cmat = _projection(x, W, attn_l, attn_r)
    el_p = jnp.pad(el[:, 0], (0, NP - N))
    er_p = jnp.pad(er[:, 0], (0, NP - N))
    c16 = cmat[0, :16]

    p, pk, dpart = _edge_kernel(src_p, dst_p, el_p, er_p, c16)
    dinv_col = _den_reduce(dpart).reshape(NP, 1)

    perm = np.zeros(DH, dtype=np.int32)
    for g in range(DH // 32):
        for i in range(16):
            perm[32 * g + 2 * i] = 32 * g + i
            perm[32 * g + 2 * i + 1] = 32 * g + 16 + i
    fhbf = fhalves.astype(jnp.bfloat16)[:, :, perm]

    (parts,) = _agg_kernel(pk, p, fhbf)
    return _epilogue(parts, dinv_col, bias)
